# baseline (speedup 1.0000x reference)
import functools

import jax
import jax.numpy as jnp
from jax import lax
from jax.experimental import pallas as pl
from jax.experimental.pallas import tpu as pltpu
from jax.experimental.pallas import tpu_sc as plsc

N = 10000
D = 128
G = 64

NC = 2
NS = 16
NW = NC * NS
LANES = 16
CH = 128
NPAD = 10240
RPT_ACC = NPAD // NS
RPT_OUT = N // NS

_f32 = jnp.float32
_i32 = jnp.int32


def _sc_mesh():
    return plsc.VectorSubcoreMesh(core_axis_name="c", subcore_axis_name="s")


@functools.lru_cache(maxsize=None)
def _deg_sc(K):
    @functools.partial(
        pl.kernel,
        out_type=jax.ShapeDtypeStruct((NC, NPAD), _f32),
        mesh=_sc_mesh(),
        scratch_types=[
            pltpu.VMEM((K, CH), _i32),
            pltpu.VMEM((CH,), _f32),
            pltpu.VMEM((RPT_ACC,), _f32),
            pltpu.VMEM_SHARED((NPAD,), _f32),
        ],
    )
    def deg_kernel(dst_hbm, out_hbm, dstv, ones_v, zero_v, acc):
        c = lax.axis_index("c")
        s = lax.axis_index("s")
        widx = c * NS + s
        pltpu.sync_copy(dst_hbm.at[widx], dstv)
        for j in range(CH // LANES):
            ones_v[pl.ds(j * LANES, LANES)] = jnp.ones((LANES,), _f32)
        for j in range(RPT_ACC // LANES):
            zero_v[pl.ds(j * LANES, LANES)] = jnp.zeros((LANES,), _f32)
        pltpu.sync_copy(zero_v, acc.at[pl.ds(s * RPT_ACC, RPT_ACC)])
        plsc.subcore_barrier()

        def step(k, carry):
            pltpu.sync_copy(ones_v, acc.at[dstv.at[k]], add=True)
            return carry

        lax.fori_loop(0, K, step, 0)
        plsc.subcore_barrier()
        pltpu.sync_copy(acc.at[pl.ds(s * RPT_ACC, RPT_ACC)],
                        out_hbm.at[c, pl.ds(s * RPT_ACC, RPT_ACC)])

    return deg_kernel


@functools.lru_cache(maxsize=None)
def _spmm_sc(K0, K1):
    KM = max(K0, K1)
    assert K1 % 2 == 0 and K1 + 2 <= KM

    @functools.partial(
        pl.kernel,
        out_type=jax.ShapeDtypeStruct((NC, NPAD, D), _f32),
        mesh=_sc_mesh(),
        scratch_types=[
            pltpu.VMEM((KM, CH), _i32),
            pltpu.VMEM((1, CH), _i32),
            pltpu.VMEM((1, CH), _i32),
            pltpu.VMEM((1, CH), _i32),
            pltpu.VMEM((1, CH), _i32),
            pltpu.VMEM((CH, D), _f32),
            pltpu.VMEM((CH, D), _f32),
            pltpu.VMEM_SHARED((NPAD, D), _f32),
            pltpu.SemaphoreType.DMA,
            pltpu.SemaphoreType.DMA,
        ],
    )
    def spmm_kernel(hw_hbm, pidx0_hbm, pidx1_hbm, out_hbm,
                    pidx, srcb, dstb, srcb1, dstb1, rows, rows1,
                    acc, gsem, gsem1):
        c = lax.axis_index("c")
        s = lax.axis_index("s")

        @pl.when(c == 0)
        def _():
            pltpu.sync_copy(pidx0_hbm.at[s], pidx)

        @pl.when(c == 1)
        def _():
            pltpu.sync_copy(pidx1_hbm.at[s], pidx)

        def zrow(r, carry):
            for j in range(D // LANES):
                rows[r, pl.ds(j * LANES, LANES)] = jnp.zeros((LANES,), _f32)
            return carry

        lax.fori_loop(0, CH, zrow, 0)
        for j in range(RPT_ACC // CH):
            pltpu.sync_copy(rows, acc.at[pl.ds(s * RPT_ACC + j * CH, CH)])
        plsc.subcore_barrier()

        def unpack(k, sb, db):
            for j in range(CH // LANES):
                v = pidx[k, pl.ds(j * LANES, LANES)]
                sb[0, pl.ds(j * LANES, LANES)] = v & 0xFFFF
                db[0, pl.ds(j * LANES, LANES)] = v >> 16

        @pl.when(c == 0)
        def _():
            def step(k, carry):
                unpack(k, srcb, dstb)
                pltpu.async_copy(hw_hbm.at[srcb.at[0]], rows, gsem).wait()
                pltpu.sync_copy(rows, acc.at[dstb.at[0]], add=True)
                return carry

            lax.fori_loop(0, K0, step, 0)

        @pl.when(c == 1)
        def _():
            unpack(0, srcb, dstb)
            pltpu.async_copy(hw_hbm.at[srcb.at[0]], rows, gsem)

            def pair(t, carry):
                k0 = 2 * t
                unpack(k0 + 1, srcb1, dstb1)
                pltpu.async_copy(hw_hbm.at[srcb1.at[0]], rows1, gsem1)
                pltpu.make_async_copy(hw_hbm.at[srcb.at[0]], rows,
                                      gsem).wait()
                pltpu.sync_copy(rows, acc.at[dstb.at[0]], add=True)
                unpack(k0 + 2, srcb, dstb)
                pltpu.async_copy(hw_hbm.at[srcb.at[0]], rows, gsem)
                pltpu.make_async_copy(hw_hbm.at[srcb1.at[0]], rows1,
                                      gsem1).wait()
                pltpu.sync_copy(rows1, acc.at[dstb1.at[0]], add=True)
                return carry

            lax.fori_loop(0, K1 // 2, pair, 0)
            pltpu.make_async_copy(hw_hbm.at[srcb.at[0]], rows, gsem).wait()

        plsc.subcore_barrier()
        pltpu.sync_copy(acc.at[pl.ds(s * RPT_ACC, RPT_ACC)],
                        out_hbm.at[c, pl.ds(s * RPT_ACC, RPT_ACC)])

    return spmm_kernel


BM = 1000


def _hw1_tc(x, W, d0, d1):
    def body(x_ref, w_ref, d0_ref, d1_ref, hw_ref, dv_ref):
        deg = d0_ref[...] + d1_ref[...]
        dinv = jnp.where(deg > 0.0, lax.rsqrt(deg), 0.0)
        dv_ref[...] = dinv
        hw = jnp.dot(x_ref[...], w_ref[...], preferred_element_type=_f32)
        hw_ref[...] = hw * dinv

    return pl.pallas_call(
        body,
        grid=(N // BM,),
        in_specs=[
            pl.BlockSpec((BM, D), lambda i: (i, 0)),
            pl.BlockSpec((D, D), lambda i: (0, 0)),
            pl.BlockSpec((BM, 1), lambda i: (i, 0)),
            pl.BlockSpec((BM, 1), lambda i: (i, 0)),
        ],
        out_specs=[
            pl.BlockSpec((BM, D), lambda i: (i, 0)),
            pl.BlockSpec((BM, 1), lambda i: (i, 0)),
        ],
        out_shape=[
            jax.ShapeDtypeStruct((N, D), _f32),
            jax.ShapeDtypeStruct((N, 1), _f32),
        ],
    )(x, W, d0, d1)


def _pool_tc(v, batch2d):
    def body(v_ref, b_ref, pool_ref):
        i = pl.program_id(0)
        onehot = (b_ref[...] == lax.broadcasted_iota(_i32, (1, G), 1)
                  ).astype(_f32)
        p = lax.dot_general(onehot, v_ref[...], (((0,), (0,)), ((), ())),
                            preferred_element_type=_f32)

        @pl.when(i == 0)
        def _():
            pool_ref[...] = p

        @pl.when(i > 0)
        def _():
            pool_ref[...] += p

    return pl.pallas_call(
        body,
        grid=(N // BM,),
        in_specs=[
            pl.BlockSpec((BM, D), lambda i: (i, 0)),
            pl.BlockSpec((BM, 1), lambda i: (i, 0)),
        ],
        out_specs=pl.BlockSpec((G, D), lambda i: (0, 0)),
        out_shape=jax.ShapeDtypeStruct((G, D), _f32),
    )(v, batch2d)


def _t_sums(p0, p1, dinv_col, bias_row):
    def body(p0_ref, p1_ref, dv_ref, b_ref, t_ref, s_ref):
        i = pl.program_id(0)
        t = (p0_ref[...] + p1_ref[...]) * dv_ref[...] + b_ref[...]
        t_ref[...] = t
        st = jnp.sum(t, axis=0, keepdims=True)
        st2 = jnp.sum(t * t, axis=0, keepdims=True)
        blk = jnp.concatenate(
            [st, st2, jnp.zeros((6, 128), _f32)], axis=0)

        @pl.when(i == 0)
        def _():
            s_ref[...] = blk

        @pl.when(i > 0)
        def _():
            s_ref[...] += blk

    return pl.pallas_call(
        body,
        grid=(N // BM,),
        in_specs=[
            pl.BlockSpec((BM, D), lambda i: (i, 0)),
            pl.BlockSpec((BM, D), lambda i: (i, 0)),
            pl.BlockSpec((BM, 1), lambda i: (i, 0)),
            pl.BlockSpec((1, D), lambda i: (0, 0)),
        ],
        out_specs=[
            pl.BlockSpec((BM, D), lambda i: (i, 0)),
            pl.BlockSpec((8, D), lambda i: (0, 0)),
        ],
        out_shape=[
            jax.ShapeDtypeStruct((N, D), _f32),
            jax.ShapeDtypeStruct((8, D), _f32),
        ],
    )(p0, p1, dinv_col, bias_row)


def _bn_stats(s_ref):
    m = s_ref[0:1, :] * (1.0 / N)
    ex2 = s_ref[1:2, :] * (1.0 / N)
    var = ex2 - m * m
    inv = lax.rsqrt(var + 1e-5)
    return m, inv


def _bn_mm(t, sums, gamma_row, beta_row, W, dinv_col):
    def body(t_ref, s_ref, g_ref, be_ref, w_ref, dv_ref, hw_ref):
        m, inv = _bn_stats(s_ref)
        h = jnp.maximum((t_ref[...] - m) * inv * g_ref[...] + be_ref[...],
                        0.0)
        hw = jnp.dot(h, w_ref[...], preferred_element_type=_f32)
        hw_ref[...] = hw * dv_ref[...]

    return pl.pallas_call(
        body,
        grid=(N // BM,),
        in_specs=[
            pl.BlockSpec((BM, D), lambda i: (i, 0)),
            pl.BlockSpec((8, D), lambda i: (0, 0)),
            pl.BlockSpec((1, D), lambda i: (0, 0)),
            pl.BlockSpec((1, D), lambda i: (0, 0)),
            pl.BlockSpec((D, D), lambda i: (0, 0)),
            pl.BlockSpec((BM, 1), lambda i: (i, 0)),
        ],
        out_specs=pl.BlockSpec((BM, D), lambda i: (i, 0)),
        out_shape=jax.ShapeDtypeStruct((N, D), _f32),
    )(t, sums, gamma_row, beta_row, W, dinv_col)


def _bn_pool(t, sums, gamma_row, beta_row, batch2d):
    def body(t_ref, s_ref, g_ref, be_ref, b_ref, pool_ref):
        i = pl.program_id(0)
        m, inv = _bn_stats(s_ref)
        h = jnp.maximum((t_ref[...] - m) * inv * g_ref[...] + be_ref[...],
                        0.0)
        onehot = (b_ref[...] == lax.broadcasted_iota(_i32, (1, G), 1)
                  ).astype(_f32)
        p = lax.dot_general(onehot, h, (((0,), (0,)), ((), ())),
                            preferred_element_type=_f32)

        @pl.when(i == 0)
        def _():
            pool_ref[...] = p

        @pl.when(i > 0)
        def _():
            pool_ref[...] += p

    return pl.pallas_call(
        body,
        grid=(N // BM,),
        in_specs=[
            pl.BlockSpec((BM, D), lambda i: (i, 0)),
            pl.BlockSpec((8, D), lambda i: (0, 0)),
            pl.BlockSpec((1, D), lambda i: (0, 0)),
            pl.BlockSpec((1, D), lambda i: (0, 0)),
            pl.BlockSpec((BM, 1), lambda i: (i, 0)),
        ],
        out_specs=pl.BlockSpec((G, D), lambda i: (0, 0)),
        out_shape=jax.ShapeDtypeStruct((G, D), _f32),
    )(t, sums, gamma_row, beta_row, batch2d)


def _bn_pool_final(t, sums, gamma_row, beta_row, batch2d,
                   px, p1, Wp0, Wp1, Wp2, b0r, b1r, b2r):
    def body(t_ref, s_ref, g_ref, be_ref, b_ref, px_ref, p1_ref,
             w0_ref, w1_ref, w2_ref, b0_ref, b1_ref, b2_ref,
             o_ref, pool_acc):
        i = pl.program_id(0)
        m, inv = _bn_stats(s_ref)
        h = jnp.maximum((t_ref[...] - m) * inv * g_ref[...] + be_ref[...],
                        0.0)
        onehot = (b_ref[...] == lax.broadcasted_iota(_i32, (1, G), 1)
                  ).astype(_f32)
        p = lax.dot_general(onehot, h, (((0,), (0,)), ((), ())),
                            preferred_element_type=_f32)

        @pl.when(i == 0)
        def _():
            pool_acc[...] = p

        @pl.when(i > 0)
        def _():
            pool_acc[...] += p

        @pl.when(i == N // BM - 1)
        def _():
            o_ref[...] = (
                jnp.dot(px_ref[...], w0_ref[...],
                        preferred_element_type=_f32)
                + jnp.dot(p1_ref[...], w1_ref[...],
                          preferred_element_type=_f32)
                + jnp.dot(pool_acc[...], w2_ref[...],
                          preferred_element_type=_f32)
                + b0_ref[...] + b1_ref[...] + b2_ref[...])

    return pl.pallas_call(
        body,
        grid=(N // BM,),
        in_specs=[
            pl.BlockSpec((BM, D), lambda i: (i, 0)),
            pl.BlockSpec((8, D), lambda i: (0, 0)),
            pl.BlockSpec((1, D), lambda i: (0, 0)),
            pl.BlockSpec((1, D), lambda i: (0, 0)),
            pl.BlockSpec((BM, 1), lambda i: (i, 0)),
            pl.BlockSpec((G, D), lambda i: (0, 0)),
            pl.BlockSpec((G, D), lambda i: (0, 0)),
            pl.BlockSpec((D, D), lambda i: (0, 0)),
            pl.BlockSpec((D, D), lambda i: (0, 0)),
            pl.BlockSpec((D, D), lambda i: (0, 0)),
            pl.BlockSpec((1, D), lambda i: (0, 0)),
            pl.BlockSpec((1, D), lambda i: (0, 0)),
            pl.BlockSpec((1, D), lambda i: (0, 0)),
        ],
        out_specs=pl.BlockSpec((G, D), lambda i: (0, 0)),
        out_shape=jax.ShapeDtypeStruct((G, D), _f32),
        scratch_shapes=[pltpu.VMEM((G, D), _f32)],
    )(t, sums, gamma_row, beta_row, batch2d, px, p1,
      Wp0, Wp1, Wp2, b0r, b1r, b2r)


def kernel(x, edge_index, batch, W1, b1, W2, b2, gamma1, beta1,
           gamma2, beta2, Wp0, bp0, Wp1, bp1, Wp2, bp2):
    E = edge_index.shape[1]
    etot = E + N
    loop = jnp.arange(N, dtype=_i32)
    src_flat = jnp.concatenate([edge_index[0].astype(_i32), loop])
    dst_flat = jnp.concatenate([edge_index[1].astype(_i32), loop])

    KD = -(-etot // (NW * CH))
    epd = NW * KD * CH
    dst_d = jnp.concatenate(
        [dst_flat, jnp.full((epd - etot,), N, _i32)]).reshape(NW, KD, CH)

    tot = -(-etot // CH)
    per_tile = -(-tot // NS)
    K0 = int(round(per_tile * 0.6235))
    K1 = per_tile - K0
    while NS * (K0 + K1) * CH < etot or K1 % 2:
        K1 += 1
    tot0 = NS * K0 * CH
    tot1 = NS * K1 * CH
    packed_flat = src_flat | (dst_flat << 16)
    packed_p = jnp.concatenate(
        [packed_flat, jnp.full((tot0 + tot1 - etot,), N << 16, _i32)])
    pidx0 = packed_p[:tot0].reshape(NS, K0, CH)
    pidx1 = packed_p[tot0:].reshape(NS, K1, CH)
    pidx1 = jnp.concatenate(
        [pidx1, jnp.full((NS, K0 - K1, CH), N << 16, _i32)], axis=1)

    degp = _deg_sc(KD)(dst_d)
    d0 = degp[0, :N].reshape(N, 1)
    d1 = degp[1, :N].reshape(N, 1)
    batch2d = batch.astype(_i32).reshape(N, 1)

    hw1, dinv_col = _hw1_tc(x, W1, d0, d1)
    s1 = _spmm_sc(K0, K1)(hw1, pidx0, pidx1)
    poolx = _pool_tc(x, batch2d)
    t1, sums1 = _t_sums(s1[0, :N], s1[1, :N], dinv_col, b1.reshape(1, D))
    hw2 = _bn_mm(t1, sums1, gamma1.reshape(1, D),
                 beta1.reshape(1, D), W2, dinv_col)
    s2 = _spmm_sc(K0, K1)(hw2, pidx0, pidx1)
    pool1 = _bn_pool(t1, sums1, gamma1.reshape(1, D),
                     beta1.reshape(1, D), batch2d)
    t2, sums2 = _t_sums(s2[0, :N], s2[1, :N], dinv_col, b2.reshape(1, D))
    return _bn_pool_final(t2, sums2, gamma2.reshape(1, D),
                          beta2.reshape(1, D), batch2d, poolx, pool1,
                          Wp0, Wp1, Wp2, bp0.reshape(1, D),
                          bp1.reshape(1, D), bp2.reshape(1, D))


# speedup vs baseline: 1.3747x; 1.3747x over previous
import functools

import jax
import jax.numpy as jnp
from jax import lax
from jax.experimental import pallas as pl
from jax.experimental.pallas import tpu as pltpu
from jax.experimental.pallas import tpu_sc as plsc

N = 10000
D = 128
G = 64

NC = 2
NS = 16
NW = NC * NS
LANES = 16
CH = 128
NPAD = 10240
RPT_ACC = NPAD // NS
RPT_OUT = N // NS

_f32 = jnp.float32
_i32 = jnp.int32


def _sc_mesh():
    return plsc.VectorSubcoreMesh(core_axis_name="c", subcore_axis_name="s")


@functools.lru_cache(maxsize=None)
def _deg_sc(K):
    @functools.partial(
        pl.kernel,
        out_type=jax.ShapeDtypeStruct((NC, NPAD), _f32),
        mesh=_sc_mesh(),
        scratch_types=[
            pltpu.VMEM((K, CH), _i32),
            pltpu.VMEM((CH,), _f32),
            pltpu.VMEM((RPT_ACC,), _f32),
            pltpu.VMEM_SHARED((NPAD,), _f32),
        ],
    )
    def deg_kernel(dst_hbm, out_hbm, dstv, ones_v, zero_v, acc):
        c = lax.axis_index("c")
        s = lax.axis_index("s")
        widx = c * NS + s
        pltpu.sync_copy(dst_hbm.at[widx], dstv)
        for j in range(CH // LANES):
            ones_v[pl.ds(j * LANES, LANES)] = jnp.ones((LANES,), _f32)
        for j in range(RPT_ACC // LANES):
            zero_v[pl.ds(j * LANES, LANES)] = jnp.zeros((LANES,), _f32)
        pltpu.sync_copy(zero_v, acc.at[pl.ds(s * RPT_ACC, RPT_ACC)])
        plsc.subcore_barrier()

        def step(k, carry):
            pltpu.sync_copy(ones_v, acc.at[dstv.at[k]], add=True)
            return carry

        lax.fori_loop(0, K, step, 0)
        plsc.subcore_barrier()
        pltpu.sync_copy(acc.at[pl.ds(s * RPT_ACC, RPT_ACC)],
                        out_hbm.at[c, pl.ds(s * RPT_ACC, RPT_ACC)])

    return deg_kernel


@functools.lru_cache(maxsize=None)
def _spmm_sc(K0, K1):
    KM = max(K0, K1)

    @functools.partial(
        pl.kernel,
        out_type=jax.ShapeDtypeStruct((NC, NPAD, D), _f32),
        mesh=_sc_mesh(),
        scratch_types=[
            pltpu.VMEM((KM, CH), _i32),
            pltpu.VMEM((1, CH), _i32),
            pltpu.VMEM((1, CH), _i32),
            pltpu.VMEM((CH, D), _f32),
            pltpu.VMEM_SHARED((NPAD, D), _f32),
            pltpu.SemaphoreType.DMA,
        ],
    )
    def spmm_kernel(hw_hbm, pidx0_hbm, pidx1_hbm, out_hbm,
                    pidx, srcb, dstb, rows, acc, gsem):
        c = lax.axis_index("c")
        s = lax.axis_index("s")

        @pl.when(c == 0)
        def _():
            pltpu.sync_copy(pidx0_hbm.at[s], pidx)

        @pl.when(c == 1)
        def _():
            pltpu.sync_copy(pidx1_hbm.at[s], pidx)

        def zrow(r, carry):
            for j in range(D // LANES):
                rows[r, pl.ds(j * LANES, LANES)] = jnp.zeros((LANES,), _f32)
            return carry

        lax.fori_loop(0, CH, zrow, 0)
        for j in range(RPT_ACC // CH):
            pltpu.sync_copy(rows, acc.at[pl.ds(s * RPT_ACC + j * CH, CH)])
        plsc.subcore_barrier()

        def step(k, carry):
            for j in range(CH // LANES):
                v = pidx[k, pl.ds(j * LANES, LANES)]
                srcb[0, pl.ds(j * LANES, LANES)] = v & 0xFFFF
                dstb[0, pl.ds(j * LANES, LANES)] = v >> 16
            pltpu.async_copy(hw_hbm.at[srcb.at[0]], rows, gsem).wait()
            pltpu.sync_copy(rows, acc.at[dstb.at[0]], add=True)
            return carry

        nk = jnp.where(c == 0, K0, K1)
        lax.fori_loop(0, nk, step, 0)
        plsc.subcore_barrier()
        pltpu.sync_copy(acc.at[pl.ds(s * RPT_ACC, RPT_ACC)],
                        out_hbm.at[c, pl.ds(s * RPT_ACC, RPT_ACC)])

    return spmm_kernel


BM = 1000


def _hw1_tc(x, W, d0, d1):
    def body(x_ref, w_ref, d0_ref, d1_ref, hw_ref, dv_ref):
        deg = d0_ref[...] + d1_ref[...]
        dinv = jnp.where(deg > 0.0, lax.rsqrt(deg), 0.0)
        dv_ref[...] = dinv
        hw = jnp.dot(x_ref[...], w_ref[...], preferred_element_type=_f32)
        hw_ref[...] = hw * dinv

    return pl.pallas_call(
        body,
        grid=(N // BM,),
        in_specs=[
            pl.BlockSpec((BM, D), lambda i: (i, 0)),
            pl.BlockSpec((D, D), lambda i: (0, 0)),
            pl.BlockSpec((BM, 1), lambda i: (i, 0)),
            pl.BlockSpec((BM, 1), lambda i: (i, 0)),
        ],
        out_specs=[
            pl.BlockSpec((BM, D), lambda i: (i, 0)),
            pl.BlockSpec((BM, 1), lambda i: (i, 0)),
        ],
        out_shape=[
            jax.ShapeDtypeStruct((N, D), _f32),
            jax.ShapeDtypeStruct((N, 1), _f32),
        ],
    )(x, W, d0, d1)


def _pool_tc(v, batch2d):
    def body(v_ref, b_ref, pool_ref):
        i = pl.program_id(0)
        onehot = (b_ref[...] == lax.broadcasted_iota(_i32, (1, G), 1)
                  ).astype(_f32)
        p = lax.dot_general(onehot, v_ref[...], (((0,), (0,)), ((), ())),
                            preferred_element_type=_f32)

        @pl.when(i == 0)
        def _():
            pool_ref[...] = p

        @pl.when(i > 0)
        def _():
            pool_ref[...] += p

    return pl.pallas_call(
        body,
        grid=(N // BM,),
        in_specs=[
            pl.BlockSpec((BM, D), lambda i: (i, 0)),
            pl.BlockSpec((BM, 1), lambda i: (i, 0)),
        ],
        out_specs=pl.BlockSpec((G, D), lambda i: (0, 0)),
        out_shape=jax.ShapeDtypeStruct((G, D), _f32),
    )(v, batch2d)


def _t_sums(p0, p1, dinv_col, bias_row):
    def body(p0_ref, p1_ref, dv_ref, b_ref, t_ref, s_ref):
        i = pl.program_id(0)
        t = (p0_ref[...] + p1_ref[...]) * dv_ref[...] + b_ref[...]
        t_ref[...] = t
        st = jnp.sum(t, axis=0, keepdims=True)
        st2 = jnp.sum(t * t, axis=0, keepdims=True)
        blk = jnp.concatenate(
            [st, st2, jnp.zeros((6, 128), _f32)], axis=0)

        @pl.when(i == 0)
        def _():
            s_ref[...] = blk

        @pl.when(i > 0)
        def _():
            s_ref[...] += blk

    return pl.pallas_call(
        body,
        grid=(N // BM,),
        in_specs=[
            pl.BlockSpec((BM, D), lambda i: (i, 0)),
            pl.BlockSpec((BM, D), lambda i: (i, 0)),
            pl.BlockSpec((BM, 1), lambda i: (i, 0)),
            pl.BlockSpec((1, D), lambda i: (0, 0)),
        ],
        out_specs=[
            pl.BlockSpec((BM, D), lambda i: (i, 0)),
            pl.BlockSpec((8, D), lambda i: (0, 0)),
        ],
        out_shape=[
            jax.ShapeDtypeStruct((N, D), _f32),
            jax.ShapeDtypeStruct((8, D), _f32),
        ],
    )(p0, p1, dinv_col, bias_row)


def _bn_stats(s_ref):
    m = s_ref[0:1, :] * (1.0 / N)
    ex2 = s_ref[1:2, :] * (1.0 / N)
    var = ex2 - m * m
    inv = lax.rsqrt(var + 1e-5)
    return m, inv


def _bn_mm(t, sums, gamma_row, beta_row, W, dinv_col):
    def body(t_ref, s_ref, g_ref, be_ref, w_ref, dv_ref, hw_ref):
        m, inv = _bn_stats(s_ref)
        h = jnp.maximum((t_ref[...] - m) * inv * g_ref[...] + be_ref[...],
                        0.0)
        hw = jnp.dot(h, w_ref[...], preferred_element_type=_f32)
        hw_ref[...] = hw * dv_ref[...]

    return pl.pallas_call(
        body,
        grid=(N // BM,),
        in_specs=[
            pl.BlockSpec((BM, D), lambda i: (i, 0)),
            pl.BlockSpec((8, D), lambda i: (0, 0)),
            pl.BlockSpec((1, D), lambda i: (0, 0)),
            pl.BlockSpec((1, D), lambda i: (0, 0)),
            pl.BlockSpec((D, D), lambda i: (0, 0)),
            pl.BlockSpec((BM, 1), lambda i: (i, 0)),
        ],
        out_specs=pl.BlockSpec((BM, D), lambda i: (i, 0)),
        out_shape=jax.ShapeDtypeStruct((N, D), _f32),
    )(t, sums, gamma_row, beta_row, W, dinv_col)


def _bn_pool(t, sums, gamma_row, beta_row, batch2d):
    def body(t_ref, s_ref, g_ref, be_ref, b_ref, pool_ref):
        i = pl.program_id(0)
        m, inv = _bn_stats(s_ref)
        h = jnp.maximum((t_ref[...] - m) * inv * g_ref[...] + be_ref[...],
                        0.0)
        onehot = (b_ref[...] == lax.broadcasted_iota(_i32, (1, G), 1)
                  ).astype(_f32)
        p = lax.dot_general(onehot, h, (((0,), (0,)), ((), ())),
                            preferred_element_type=_f32)

        @pl.when(i == 0)
        def _():
            pool_ref[...] = p

        @pl.when(i > 0)
        def _():
            pool_ref[...] += p

    return pl.pallas_call(
        body,
        grid=(N // BM,),
        in_specs=[
            pl.BlockSpec((BM, D), lambda i: (i, 0)),
            pl.BlockSpec((8, D), lambda i: (0, 0)),
            pl.BlockSpec((1, D), lambda i: (0, 0)),
            pl.BlockSpec((1, D), lambda i: (0, 0)),
            pl.BlockSpec((BM, 1), lambda i: (i, 0)),
        ],
        out_specs=pl.BlockSpec((G, D), lambda i: (0, 0)),
        out_shape=jax.ShapeDtypeStruct((G, D), _f32),
    )(t, sums, gamma_row, beta_row, batch2d)


def _bn_pool_final(t, sums, gamma_row, beta_row, batch2d,
                   px, p1, Wp0, Wp1, Wp2, b0r, b1r, b2r):
    def body(t_ref, s_ref, g_ref, be_ref, b_ref, px_ref, p1_ref,
             w0_ref, w1_ref, w2_ref, b0_ref, b1_ref, b2_ref,
             o_ref, pool_acc):
        i = pl.program_id(0)
        m, inv = _bn_stats(s_ref)
        h = jnp.maximum((t_ref[...] - m) * inv * g_ref[...] + be_ref[...],
                        0.0)
        onehot = (b_ref[...] == lax.broadcasted_iota(_i32, (1, G), 1)
                  ).astype(_f32)
        p = lax.dot_general(onehot, h, (((0,), (0,)), ((), ())),
                            preferred_element_type=_f32)

        @pl.when(i == 0)
        def _():
            pool_acc[...] = p

        @pl.when(i > 0)
        def _():
            pool_acc[...] += p

        @pl.when(i == N // BM - 1)
        def _():
            o_ref[...] = (
                jnp.dot(px_ref[...], w0_ref[...],
                        preferred_element_type=_f32)
                + jnp.dot(p1_ref[...], w1_ref[...],
                          preferred_element_type=_f32)
                + jnp.dot(pool_acc[...], w2_ref[...],
                          preferred_element_type=_f32)
                + b0_ref[...] + b1_ref[...] + b2_ref[...])

    return pl.pallas_call(
        body,
        grid=(N // BM,),
        in_specs=[
            pl.BlockSpec((BM, D), lambda i: (i, 0)),
            pl.BlockSpec((8, D), lambda i: (0, 0)),
            pl.BlockSpec((1, D), lambda i: (0, 0)),
            pl.BlockSpec((1, D), lambda i: (0, 0)),
            pl.BlockSpec((BM, 1), lambda i: (i, 0)),
            pl.BlockSpec((G, D), lambda i: (0, 0)),
            pl.BlockSpec((G, D), lambda i: (0, 0)),
            pl.BlockSpec((D, D), lambda i: (0, 0)),
            pl.BlockSpec((D, D), lambda i: (0, 0)),
            pl.BlockSpec((D, D), lambda i: (0, 0)),
            pl.BlockSpec((1, D), lambda i: (0, 0)),
            pl.BlockSpec((1, D), lambda i: (0, 0)),
            pl.BlockSpec((1, D), lambda i: (0, 0)),
        ],
        out_specs=pl.BlockSpec((G, D), lambda i: (0, 0)),
        out_shape=jax.ShapeDtypeStruct((G, D), _f32),
        scratch_shapes=[pltpu.VMEM((G, D), _f32)],
    )(t, sums, gamma_row, beta_row, batch2d, px, p1,
      Wp0, Wp1, Wp2, b0r, b1r, b2r)


def kernel(x, edge_index, batch, W1, b1, W2, b2, gamma1, beta1,
           gamma2, beta2, Wp0, bp0, Wp1, bp1, Wp2, bp2):
    E = edge_index.shape[1]
    etot = E + N
    loop = jnp.arange(N, dtype=_i32)
    src_flat = jnp.concatenate([edge_index[0].astype(_i32), loop])
    dst_flat = jnp.concatenate([edge_index[1].astype(_i32), loop])

    KD = -(-etot // (NW * CH))
    epd = NW * KD * CH
    dst_d = jnp.concatenate(
        [dst_flat, jnp.full((epd - etot,), N, _i32)]).reshape(NW, KD, CH)

    tot = -(-etot // CH)
    per_tile = -(-tot // NS)
    K0 = int(round(per_tile * 0.6235))
    K1 = per_tile - K0
    while NS * (K0 + K1) * CH < etot:
        K1 += 1
    tot0 = NS * K0 * CH
    tot1 = NS * K1 * CH
    packed_flat = src_flat | (dst_flat << 16)
    packed_p = jnp.concatenate(
        [packed_flat, jnp.full((tot0 + tot1 - etot,), N << 16, _i32)])
    pidx0 = packed_p[:tot0].reshape(NS, K0, CH)
    pidx1 = packed_p[tot0:].reshape(NS, K1, CH)
    pidx1 = jnp.concatenate(
        [pidx1, jnp.full((NS, K0 - K1, CH), N << 16, _i32)], axis=1)

    degp = _deg_sc(KD)(dst_d)
    d0 = degp[0, :N].reshape(N, 1)
    d1 = degp[1, :N].reshape(N, 1)
    batch2d = batch.astype(_i32).reshape(N, 1)

    hw1, dinv_col = _hw1_tc(x, W1, d0, d1)
    s1 = _spmm_sc(K0, K1)(hw1, pidx0, pidx1)
    poolx = _pool_tc(x, batch2d)
    t1, sums1 = _t_sums(s1[0, :N], s1[1, :N], dinv_col, b1.reshape(1, D))
    hw2 = _bn_mm(t1, sums1, gamma1.reshape(1, D),
                 beta1.reshape(1, D), W2, dinv_col)
    s2 = _spmm_sc(K0, K1)(hw2, pidx0, pidx1)
    pool1 = _bn_pool(t1, sums1, gamma1.reshape(1, D),
                     beta1.reshape(1, D), batch2d)
    t2, sums2 = _t_sums(s2[0, :N], s2[1, :N], dinv_col, b2.reshape(1, D))
    return _bn_pool_final(t2, sums2, gamma2.reshape(1, D),
                          beta2.reshape(1, D), batch2d, poolx, pool1,
                          Wp0, Wp1, Wp2, bp0.reshape(1, D),
                          bp1.reshape(1, D), bp2.reshape(1, D))
